# Initial kernel scaffold; baseline (speedup 1.0000x reference)
import jax
import jax.numpy as jnp
from jax.experimental import pallas as pl


def kernel(x, sos, table):
    raise NotImplementedError("write your pallas kernel here")


# speedup vs baseline: 6.5818x; 6.5818x over previous
import functools

import jax
import jax.numpy as jnp
from jax import lax
from jax.experimental import pallas as pl
from jax.experimental.pallas import tpu as pltpu
from jax.experimental.pallas import tpu_sc as plsc

_B, _L, _S = 1024, 50, 32
_SUB = 32
_T = _L + 1
_RPB = _L * _S
_NC, _NS = 2, 16
_NW = _NC * _NS
_BPW = _B // _NW


def _make_sc_kernel():
    mesh = plsc.VectorSubcoreMesh(core_axis_name="c", subcore_axis_name="s")

    @functools.partial(
        pl.kernel,
        mesh=mesh,
        out_type=jax.ShapeDtypeStruct((_B * _T * _S, _SUB), jnp.float32),
        scratch_types=[
            pltpu.VMEM((_RPB,), jnp.int32),
            pltpu.VMEM((_RPB, _SUB), jnp.float32),
            pltpu.VMEM((_S, _SUB), jnp.float32),
            pltpu.SemaphoreType.DMA,
        ],
        compiler_params=pltpu.CompilerParams(use_tc_tiling_on_sc=False),
    )
    def k(x_hbm, sos_hbm, table_hbm, out_hbm, idx_v, rows_v, sos_v, sem):
        wid = lax.axis_index("s") * _NC + lax.axis_index("c")
        pltpu.sync_copy(sos_hbm, sos_v)

        def body(i, carry):
            b = wid * _BPW + i
            pltpu.sync_copy(x_hbm.at[b], idx_v)
            pltpu.async_copy(table_hbm.at[idx_v], rows_v, sem).wait()
            base = b * (_T * _S)
            pltpu.sync_copy(sos_v, out_hbm.at[pl.ds(base, _S)])
            pltpu.sync_copy(rows_v, out_hbm.at[pl.ds(base + _S, _RPB)])
            return carry

        lax.fori_loop(0, _BPW, body, 0)

    return k


_sc_kernel = _make_sc_kernel()


def kernel(x, sos, table):
    x_flat = x.reshape(_B, _L * _S).astype(jnp.int32)
    sos_rows = sos.reshape(_S, _SUB)
    out = _sc_kernel(x_flat, sos_rows, table)
    return out.reshape(_B, _T, _S * _SUB)


# speedup vs baseline: 6.6271x; 1.0069x over previous
import functools

import jax
import jax.numpy as jnp
from jax import lax
from jax.experimental import pallas as pl
from jax.experimental.pallas import tpu as pltpu
from jax.experimental.pallas import tpu_sc as plsc

_B, _L, _S = 1024, 50, 32
_SUB = 32
_T = _L + 1
_RPB = _L * _S
_BROWS = _T * _S
_NC, _NS = 2, 16
_NW = _NC * _NS
_BPW = _B // _NW


def _make_sc_kernel():
    mesh = plsc.VectorSubcoreMesh(core_axis_name="c", subcore_axis_name="s")

    @functools.partial(
        pl.kernel,
        mesh=mesh,
        out_type=jax.ShapeDtypeStruct((_B * _BROWS, _SUB), jnp.float32),
        scratch_types=[
            pltpu.VMEM((_RPB,), jnp.int32),
            pltpu.VMEM((_RPB,), jnp.int32),
            pltpu.VMEM((_BROWS, _SUB), jnp.float32),
            pltpu.VMEM((_BROWS, _SUB), jnp.float32),
            pltpu.SemaphoreType.DMA,
            pltpu.SemaphoreType.DMA,
            pltpu.SemaphoreType.DMA,
            pltpu.SemaphoreType.DMA,
        ],
        compiler_params=pltpu.CompilerParams(use_tc_tiling_on_sc=False),
    )
    def k(x_hbm, sos_hbm, table_hbm, out_hbm,
          idx_a, idx_b, rows_a, rows_b, gsem_a, gsem_b, isem_a, isem_b):
        wid = lax.axis_index("s") * _NC + lax.axis_index("c")
        b0 = wid * _BPW

        pltpu.sync_copy(sos_hbm, rows_a.at[pl.ds(0, _S)])
        pltpu.sync_copy(sos_hbm, rows_b.at[pl.ds(0, _S)])

        def idx_copy(slot_ref, sem, i):
            return pltpu.make_async_copy(x_hbm.at[b0 + i], slot_ref, sem)

        def gather(idx_ref, rows_ref, sem):
            return pltpu.make_async_copy(
                table_hbm.at[idx_ref], rows_ref.at[pl.ds(_S, _RPB)], sem)

        def store(rows_ref, sem, i):
            return pltpu.make_async_copy(
                rows_ref, out_hbm.at[pl.ds((b0 + i) * _BROWS, _BROWS)], sem)

        idx_copy(idx_a, isem_a, 0).start()
        idx_copy(idx_b, isem_b, 1).start()
        idx_copy(idx_a, isem_a, 0).wait()
        gather(idx_a, rows_a, gsem_a).start()

        def body(j, carry):
            ia = 2 * j
            idx_copy(idx_b, isem_b, ia + 1).wait()
            gather(idx_b, rows_b, gsem_b).start()
            gather(idx_a, rows_a, gsem_a).wait()
            store(rows_a, gsem_a, ia).start()
            store(rows_a, gsem_a, ia).wait()

            @pl.when(j < _BPW // 2 - 1)
            def _():
                idx_copy(idx_a, isem_a, ia + 2).start()
                idx_copy(idx_a, isem_a, ia + 2).wait()
                gather(idx_a, rows_a, gsem_a).start()
                idx_copy(idx_b, isem_b, ia + 3).start()

            gather(idx_b, rows_b, gsem_b).wait()
            store(rows_b, gsem_b, ia + 1).start()
            store(rows_b, gsem_b, ia + 1).wait()
            return carry

        lax.fori_loop(0, _BPW // 2, body, 0)

    return k


_sc_kernel = _make_sc_kernel()


def kernel(x, sos, table):
    x_flat = x.reshape(_B, _L * _S).astype(jnp.int32)
    sos_rows = sos.reshape(_S, _SUB)
    out = _sc_kernel(x_flat, sos_rows, table)
    return out.reshape(_B, _T, _S * _SUB)


# speedup vs baseline: 13.5921x; 2.0510x over previous
import functools

import jax
import jax.numpy as jnp
from jax import lax
from jax.experimental import pallas as pl
from jax.experimental.pallas import tpu as pltpu
from jax.experimental.pallas import tpu_sc as plsc

_B, _L, _S = 1024, 50, 32
_SUB = 32
_T = _L + 1
_RPB = _L * _S
_BROWS = _T * _S
_NC, _NS = 2, 16
_NW = _NC * _NS
_BPW = _B // _NW


def _make_sc_kernel():
    mesh = plsc.VectorSubcoreMesh(core_axis_name="c", subcore_axis_name="s")

    @functools.partial(
        pl.kernel,
        mesh=mesh,
        out_type=jax.ShapeDtypeStruct((_B * _BROWS, _SUB), jnp.float32),
        scratch_types=[
            pltpu.VMEM((_RPB,), jnp.int32),
            pltpu.VMEM((_RPB,), jnp.int32),
            pltpu.VMEM((_BROWS, _SUB), jnp.float32),
            pltpu.VMEM((_BROWS, _SUB), jnp.float32),
            pltpu.VMEM_SHARED((386, _SUB), jnp.float32),
            pltpu.SemaphoreType.DMA,
            pltpu.SemaphoreType.DMA,
            pltpu.SemaphoreType.DMA,
            pltpu.SemaphoreType.DMA,
        ],
        compiler_params=pltpu.CompilerParams(use_tc_tiling_on_sc=False),
    )
    def k(x_hbm, sos_hbm, table_hbm, out_hbm,
          idx_a, idx_b, rows_a, rows_b, table_v, gsem_a, gsem_b, isem_a, isem_b):
        wid = lax.axis_index("s") * _NC + lax.axis_index("c")
        b0 = wid * _BPW

        @pl.when(lax.axis_index("s") == 0)
        def _():
            pltpu.sync_copy(table_hbm, table_v)

        plsc.subcore_barrier()
        pltpu.sync_copy(sos_hbm, rows_a.at[pl.ds(0, _S)])
        pltpu.sync_copy(sos_hbm, rows_b.at[pl.ds(0, _S)])

        def idx_copy(slot_ref, sem, i):
            return pltpu.make_async_copy(x_hbm.at[b0 + i], slot_ref, sem)

        def gather(idx_ref, rows_ref, sem):
            return pltpu.make_async_copy(
                table_v.at[idx_ref], rows_ref.at[pl.ds(_S, _RPB)], sem)

        def store(rows_ref, sem, i):
            return pltpu.make_async_copy(
                rows_ref, out_hbm.at[pl.ds((b0 + i) * _BROWS, _BROWS)], sem)

        idx_copy(idx_a, isem_a, 0).start()
        idx_copy(idx_b, isem_b, 1).start()
        idx_copy(idx_a, isem_a, 0).wait()
        gather(idx_a, rows_a, gsem_a).start()

        def body(j, carry):
            ia = 2 * j
            idx_copy(idx_b, isem_b, ia + 1).wait()
            gather(idx_b, rows_b, gsem_b).start()
            gather(idx_a, rows_a, gsem_a).wait()
            store(rows_a, gsem_a, ia).start()
            store(rows_a, gsem_a, ia).wait()

            @pl.when(j < _BPW // 2 - 1)
            def _():
                idx_copy(idx_a, isem_a, ia + 2).start()
                idx_copy(idx_a, isem_a, ia + 2).wait()
                gather(idx_a, rows_a, gsem_a).start()
                idx_copy(idx_b, isem_b, ia + 3).start()

            gather(idx_b, rows_b, gsem_b).wait()
            store(rows_b, gsem_b, ia + 1).start()
            store(rows_b, gsem_b, ia + 1).wait()
            return carry

        lax.fori_loop(0, _BPW // 2, body, 0)

    return k


_sc_kernel = _make_sc_kernel()


def kernel(x, sos, table):
    x_flat = x.reshape(_B, _L * _S).astype(jnp.int32)
    sos_rows = sos.reshape(_S, _SUB)
    out = _sc_kernel(x_flat, sos_rows, table)
    return out.reshape(_B, _T, _S * _SUB)
